# baseline (speedup 1.0000x reference)
import jax
import jax.numpy as jnp
from jax import lax
from jax.experimental import pallas as pl
from jax.experimental.pallas import tpu as pltpu
from jax.experimental.pallas import tpu_sc as plsc

NC = 2
NS = 16
NW = NC * NS
LANES = 16

N_VERT = 1_000_000
BATCH = 16384
OUT_DIM = 32
PACK = 128 // OUT_DIM
L_STEP = 1024
N_STEP = -(-N_VERT // L_STEP)
N_BLK = N_STEP * (L_STEP // PACK)
B_PER_W = BATCH // NW
CHUNK = 128
N_CHUNKS = B_PER_W // CHUNK
NBUF = 2

Q = L_STEP // PACK


def _relayout_body(in_ref, out_ref):
    x = in_ref[...]
    for a in range(PACK):
        out_ref[:, 32 * a:32 * a + 32] = x[:, 256 * a:256 * a + 256].T


def _relayout(tT):
    return pl.pallas_call(
        _relayout_body,
        grid=(N_STEP,),
        in_specs=[pl.BlockSpec((32, L_STEP), lambda s: (0, s))],
        out_specs=pl.BlockSpec((Q, 128), lambda s: (s, 0)),
        out_shape=jax.ShapeDtypeStruct((N_BLK, 128), jnp.float32),
    )(tT)


def _sc_kernel(i_hbm, j_hbm, u_hbm, v_hbm, out_hbm,
               raw_u, raw_v, blk_u, blk_v, u_buf, v_buf, out_v, sem):
    wid = lax.axis_index("s") * NC + lax.axis_index("c")
    base = wid * B_PER_W

    pltpu.sync_copy(i_hbm.at[wid], raw_u)
    pltpu.sync_copy(j_hbm.at[wid], raw_v)
    for k in range(N_CHUNKS):
        for t in range(CHUNK // LANES):
            s = pl.ds(t * LANES, LANES)
            ru = raw_u[k, s]
            rv = raw_v[k, s]
            blk_u[k, s] = ((ru >> 10) << 8) | (ru & 255)
            blk_v[k, s] = ((rv >> 10) << 8) | (rv & 255)

    def fire(k, buf):
        cu = pltpu.async_copy(u_hbm.at[blk_u.at[k]], u_buf.at[buf], sem)
        cv = pltpu.async_copy(v_hbm.at[blk_v.at[k]], v_buf.at[buf], sem)
        return cu, cv

    def compute(k, buf):
        def gbody(g, _):
            s = pl.ds(g * LANES, LANES)
            ru = raw_u[k, s]
            rv = raw_v[k, s]
            cu0 = ((ru >> 8) & 3) << 5
            cv0 = ((rv >> 8) & 3) << 5
            rows = g * LANES + lax.iota(jnp.int32, LANES)
            acc = jnp.zeros((LANES,), jnp.float32)
            for d in range(OUT_DIM):
                ud = plsc.load_gather(u_buf.at[buf], [rows, cu0 + d])
                vd = plsc.load_gather(v_buf.at[buf], [rows, cv0 + d])
                acc = acc + ud * vd
            out_v[pl.ds(k * CHUNK + g * LANES, LANES)] = acc
            return 0

        lax.fori_loop(0, CHUNK // LANES, gbody, 0)

    copies = [None] * N_CHUNKS
    copies[0] = fire(0, 0)
    for k in range(N_CHUNKS):
        if k + 1 < N_CHUNKS:
            copies[k + 1] = fire(k + 1, (k + 1) % NBUF)
        cu, cv = copies[k]
        cu.wait()
        cv.wait()
        compute(k, k % NBUF)

    pltpu.sync_copy(out_v, out_hbm.at[pl.ds(base, B_PER_W)])


@jax.jit
def _run(i3, j3, u_emb, v_emb):
    u2 = _relayout(u_emb.T)
    v2 = _relayout(v_emb.T)
    mesh = plsc.VectorSubcoreMesh(
        core_axis_name="c", subcore_axis_name="s",
        num_cores=NC, num_subcores=NS)
    f = pl.kernel(
        _sc_kernel,
        out_type=jax.ShapeDtypeStruct((BATCH,), jnp.float32),
        mesh=mesh,
        compiler_params=pltpu.CompilerParams(needs_layout_passes=False),
        scratch_types=[
            pltpu.VMEM((N_CHUNKS, CHUNK), jnp.int32),
            pltpu.VMEM((N_CHUNKS, CHUNK), jnp.int32),
            pltpu.VMEM((N_CHUNKS, CHUNK), jnp.int32),
            pltpu.VMEM((N_CHUNKS, CHUNK), jnp.int32),
            pltpu.VMEM((NBUF, CHUNK, 128), jnp.float32),
            pltpu.VMEM((NBUF, CHUNK, 128), jnp.float32),
            pltpu.VMEM((B_PER_W,), jnp.float32),
            pltpu.SemaphoreType.DMA,
        ],
    )
    return f(i3, j3, u2, v2)


def kernel(i, j, u_emb, v_emb):
    i3 = i.astype(jnp.int32).reshape(NW, N_CHUNKS, CHUNK)
    j3 = j.astype(jnp.int32).reshape(NW, N_CHUNKS, CHUNK)
    return _run(i3, j3, u_emb, v_emb)


# speedup vs baseline: 1.1734x; 1.1734x over previous
import jax
import jax.numpy as jnp
from jax import lax
from jax.experimental import pallas as pl
from jax.experimental.pallas import tpu as pltpu
from jax.experimental.pallas import tpu_sc as plsc

NC = 2
NS = 16
NW = NC * NS
LANES = 16

N_VERT = 1_000_000
BATCH = 16384
OUT_DIM = 32
PACK = 128 // OUT_DIM
L_STEP = 4096
N_STEP = -(-N_VERT // L_STEP)
N_BLK = N_STEP * (L_STEP // PACK)
B_PER_W = BATCH // NW
CHUNK = 128
N_CHUNKS = B_PER_W // CHUNK
NBUF = 2

Q = L_STEP // PACK


def _relayout_body(in_ref, out_ref):
    x = in_ref[...]
    eye = jnp.eye(32, dtype=jnp.float32)
    for a in range(PACK):
        xa = x[:, Q * a:Q * a + Q]
        out_ref[:, 32 * a:32 * a + 32] = lax.dot_general(
            xa, eye, (((0,), (0,)), ((), ())),
            precision=lax.Precision.HIGHEST)


def _relayout(tT):
    return pl.pallas_call(
        _relayout_body,
        grid=(N_STEP,),
        in_specs=[pl.BlockSpec((32, L_STEP), lambda s: (0, s))],
        out_specs=pl.BlockSpec((Q, 128), lambda s: (s, 0)),
        out_shape=jax.ShapeDtypeStruct((N_BLK, 128), jnp.float32),
    )(tT)


def _sc_kernel(i_hbm, j_hbm, u_hbm, v_hbm, out_hbm,
               raw_u, raw_v, blk_u, blk_v, u_buf, v_buf, out_v, sem):
    wid = lax.axis_index("s") * NC + lax.axis_index("c")
    base = wid * B_PER_W

    pltpu.sync_copy(i_hbm.at[wid], raw_u)
    pltpu.sync_copy(j_hbm.at[wid], raw_v)
    for k in range(N_CHUNKS):
        for t in range(CHUNK // LANES):
            s = pl.ds(t * LANES, LANES)
            ru = raw_u[k, s]
            rv = raw_v[k, s]
            blk_u[k, s] = ((ru >> 12) << 10) | (ru & 1023)
            blk_v[k, s] = ((rv >> 12) << 10) | (rv & 1023)

    def fire(k, buf):
        cu = pltpu.async_copy(u_hbm.at[blk_u.at[k]], u_buf.at[buf], sem)
        cv = pltpu.async_copy(v_hbm.at[blk_v.at[k]], v_buf.at[buf], sem)
        return cu, cv

    def compute(k, buf):
        def gbody(g, _):
            s = pl.ds(g * LANES, LANES)
            ru = raw_u[k, s]
            rv = raw_v[k, s]
            cu0 = ((ru >> 10) & 3) << 5
            cv0 = ((rv >> 10) & 3) << 5
            rows = g * LANES + lax.iota(jnp.int32, LANES)
            acc = jnp.zeros((LANES,), jnp.float32)
            for d in range(OUT_DIM):
                ud = plsc.load_gather(u_buf.at[buf], [rows, cu0 + d])
                vd = plsc.load_gather(v_buf.at[buf], [rows, cv0 + d])
                acc = acc + ud * vd
            out_v[pl.ds(k * CHUNK + g * LANES, LANES)] = acc
            return 0

        lax.fori_loop(0, CHUNK // LANES, gbody, 0)

    copies = [None] * N_CHUNKS
    copies[0] = fire(0, 0)
    for k in range(N_CHUNKS):
        if k + 1 < N_CHUNKS:
            copies[k + 1] = fire(k + 1, (k + 1) % NBUF)
        cu, cv = copies[k]
        cu.wait()
        cv.wait()
        compute(k, k % NBUF)

    pltpu.sync_copy(out_v, out_hbm.at[pl.ds(base, B_PER_W)])


@jax.jit
def _run(i3, j3, u_emb, v_emb):
    u2 = _relayout(u_emb.T)
    v2 = _relayout(v_emb.T)
    mesh = plsc.VectorSubcoreMesh(
        core_axis_name="c", subcore_axis_name="s",
        num_cores=NC, num_subcores=NS)
    f = pl.kernel(
        _sc_kernel,
        out_type=jax.ShapeDtypeStruct((BATCH,), jnp.float32),
        mesh=mesh,
        compiler_params=pltpu.CompilerParams(needs_layout_passes=False),
        scratch_types=[
            pltpu.VMEM((N_CHUNKS, CHUNK), jnp.int32),
            pltpu.VMEM((N_CHUNKS, CHUNK), jnp.int32),
            pltpu.VMEM((N_CHUNKS, CHUNK), jnp.int32),
            pltpu.VMEM((N_CHUNKS, CHUNK), jnp.int32),
            pltpu.VMEM((NBUF, CHUNK, 128), jnp.float32),
            pltpu.VMEM((NBUF, CHUNK, 128), jnp.float32),
            pltpu.VMEM((B_PER_W,), jnp.float32),
            pltpu.SemaphoreType.DMA,
        ],
    )
    return f(i3, j3, u2, v2)


def kernel(i, j, u_emb, v_emb):
    i3 = i.astype(jnp.int32).reshape(NW, N_CHUNKS, CHUNK)
    j3 = j.astype(jnp.int32).reshape(NW, N_CHUNKS, CHUNK)
    return _run(i3, j3, u_emb, v_emb)


# speedup vs baseline: 3.3676x; 2.8700x over previous
import jax
import jax.numpy as jnp
from jax import lax
from jax.experimental import pallas as pl
from jax.experimental.pallas import tpu as pltpu
from jax.experimental.pallas import tpu_sc as plsc

NC = 2
NS = 16
NW = NC * NS
LANES = 16

N_VERT = 1_000_000
BATCH = 16384
OUT_DIM = 32
PACK = 128 // OUT_DIM
L_STEP = 16384
N_STEP = -(-N_VERT // L_STEP)
N_BLK = N_STEP * (L_STEP // PACK)
B_PER_W = BATCH // NW
CHUNK = 128
N_CHUNKS = B_PER_W // CHUNK
NBUF = 2

Q = L_STEP // PACK


def _relayout_body(in_ref, out_ref):
    x = in_ref[...]
    eye = jnp.eye(32, dtype=jnp.bfloat16)
    xb = x.astype(jnp.bfloat16)
    for a in range(PACK):
        xa = xb[:, Q * a:Q * a + Q]
        out_ref[:, 32 * a:32 * a + 32] = lax.dot_general(
            xa, eye, (((0,), (0,)), ((), ())),
            preferred_element_type=jnp.float32)


def _relayout(tT):
    return pl.pallas_call(
        _relayout_body,
        grid=(N_STEP,),
        in_specs=[pl.BlockSpec((32, L_STEP), lambda s: (0, s))],
        out_specs=pl.BlockSpec((Q, 128), lambda s: (s, 0)),
        out_shape=jax.ShapeDtypeStruct((N_BLK, 128), jnp.float32),
    )(tT)


def _sc_kernel(i_hbm, j_hbm, u_hbm, v_hbm, out_hbm,
               raw_u, raw_v, blk_u, blk_v, u_buf, v_buf, out_v, sem):
    wid = lax.axis_index("s") * NC + lax.axis_index("c")
    base = wid * B_PER_W

    pltpu.sync_copy(i_hbm.at[wid], raw_u)
    pltpu.sync_copy(j_hbm.at[wid], raw_v)
    for k in range(N_CHUNKS):
        for t in range(CHUNK // LANES):
            s = pl.ds(t * LANES, LANES)
            ru = raw_u[k, s]
            rv = raw_v[k, s]
            blk_u[k, s] = ((ru >> 14) << 12) | (ru & (Q - 1))
            blk_v[k, s] = ((rv >> 14) << 12) | (rv & (Q - 1))

    def fire(k, buf):
        cu = pltpu.async_copy(u_hbm.at[blk_u.at[k]], u_buf.at[buf], sem)
        cv = pltpu.async_copy(v_hbm.at[blk_v.at[k]], v_buf.at[buf], sem)
        return cu, cv

    def compute(k, buf):
        def gbody(g, _):
            s = pl.ds(g * LANES, LANES)
            ru = raw_u[k, s]
            rv = raw_v[k, s]
            cu0 = ((ru >> 12) & 3) << 5
            cv0 = ((rv >> 12) & 3) << 5
            rows = g * LANES + lax.iota(jnp.int32, LANES)
            acc = jnp.zeros((LANES,), jnp.float32)
            for d in range(OUT_DIM):
                ud = plsc.load_gather(u_buf.at[buf], [rows, cu0 + d])
                vd = plsc.load_gather(v_buf.at[buf], [rows, cv0 + d])
                acc = acc + ud * vd
            out_v[pl.ds(k * CHUNK + g * LANES, LANES)] = acc
            return 0

        lax.fori_loop(0, CHUNK // LANES, gbody, 0)

    copies = [None] * N_CHUNKS
    copies[0] = fire(0, 0)
    for k in range(N_CHUNKS):
        if k + 1 < N_CHUNKS:
            copies[k + 1] = fire(k + 1, (k + 1) % NBUF)
        cu, cv = copies[k]
        cu.wait()
        cv.wait()
        compute(k, k % NBUF)

    pltpu.sync_copy(out_v, out_hbm.at[pl.ds(base, B_PER_W)])


@jax.jit
def _run(i3, j3, u_emb, v_emb):
    u2 = _relayout(u_emb.T)
    v2 = _relayout(v_emb.T)
    mesh = plsc.VectorSubcoreMesh(
        core_axis_name="c", subcore_axis_name="s",
        num_cores=NC, num_subcores=NS)
    f = pl.kernel(
        _sc_kernel,
        out_type=jax.ShapeDtypeStruct((BATCH,), jnp.float32),
        mesh=mesh,
        compiler_params=pltpu.CompilerParams(needs_layout_passes=False),
        scratch_types=[
            pltpu.VMEM((N_CHUNKS, CHUNK), jnp.int32),
            pltpu.VMEM((N_CHUNKS, CHUNK), jnp.int32),
            pltpu.VMEM((N_CHUNKS, CHUNK), jnp.int32),
            pltpu.VMEM((N_CHUNKS, CHUNK), jnp.int32),
            pltpu.VMEM((NBUF, CHUNK, 128), jnp.float32),
            pltpu.VMEM((NBUF, CHUNK, 128), jnp.float32),
            pltpu.VMEM((B_PER_W,), jnp.float32),
            pltpu.SemaphoreType.DMA,
        ],
    )
    return f(i3, j3, u2, v2)


def kernel(i, j, u_emb, v_emb):
    i3 = i.astype(jnp.int32).reshape(NW, N_CHUNKS, CHUNK)
    j3 = j.astype(jnp.int32).reshape(NW, N_CHUNKS, CHUNK)
    return _run(i3, j3, u_emb, v_emb)


# speedup vs baseline: 5.6749x; 1.6852x over previous
import jax
import jax.numpy as jnp
from jax import lax
from jax.experimental import pallas as pl
from jax.experimental.pallas import tpu as pltpu
from jax.experimental.pallas import tpu_sc as plsc

NC = 2
NS = 16
NW = NC * NS
LANES = 16

N_VERT = 1_000_000
BATCH = 16384
OUT_DIM = 32
PACK = 128 // OUT_DIM
L_STEP = 16384
N_STEP = -(-N_VERT // L_STEP)
N_BLK = N_STEP * (L_STEP // PACK)
B_PER_W = BATCH // NW
CHUNK = 128
N_CHUNKS = B_PER_W // CHUNK
NBUF = 2

Q = L_STEP // PACK


def _relayout_body(in_ref, out_ref):
    x = in_ref[...]
    x4 = jnp.concatenate([x[:, Q * a:Q * a + Q] for a in range(PACK)], axis=0)
    out_ref[...] = x4.T


def _relayout(tT):
    return pl.pallas_call(
        _relayout_body,
        grid=(N_STEP,),
        in_specs=[pl.BlockSpec((32, L_STEP), lambda s: (0, s))],
        out_specs=pl.BlockSpec((Q, 128), lambda s: (s, 0)),
        out_shape=jax.ShapeDtypeStruct((N_BLK, 128), jnp.float32),
    )(tT)


def _sc_kernel(i_hbm, j_hbm, u_hbm, v_hbm, out_hbm,
               raw_u, raw_v, blk_u, blk_v, u_buf, v_buf, out_v, sem):
    wid = lax.axis_index("s") * NC + lax.axis_index("c")
    base = wid * B_PER_W

    pltpu.sync_copy(i_hbm.at[wid], raw_u)
    pltpu.sync_copy(j_hbm.at[wid], raw_v)
    for k in range(N_CHUNKS):
        for t in range(CHUNK // LANES):
            s = pl.ds(t * LANES, LANES)
            ru = raw_u[k, s]
            rv = raw_v[k, s]
            blk_u[k, s] = ((ru >> 14) << 12) | (ru & (Q - 1))
            blk_v[k, s] = ((rv >> 14) << 12) | (rv & (Q - 1))

    def fire(k, buf):
        cu = pltpu.async_copy(u_hbm.at[blk_u.at[k]], u_buf.at[buf], sem)
        cv = pltpu.async_copy(v_hbm.at[blk_v.at[k]], v_buf.at[buf], sem)
        return cu, cv

    def compute(k, buf):
        def gbody(g, _):
            s = pl.ds(g * LANES, LANES)
            ru = raw_u[k, s]
            rv = raw_v[k, s]
            cu0 = ((ru >> 12) & 3) << 5
            cv0 = ((rv >> 12) & 3) << 5
            rows = g * LANES + lax.iota(jnp.int32, LANES)
            acc = jnp.zeros((LANES,), jnp.float32)
            for d in range(OUT_DIM):
                ud = plsc.load_gather(u_buf.at[buf], [rows, cu0 + d])
                vd = plsc.load_gather(v_buf.at[buf], [rows, cv0 + d])
                acc = acc + ud * vd
            out_v[pl.ds(k * CHUNK + g * LANES, LANES)] = acc
            return 0

        lax.fori_loop(0, CHUNK // LANES, gbody, 0)

    copies = [None] * N_CHUNKS
    copies[0] = fire(0, 0)
    for k in range(N_CHUNKS):
        if k + 1 < N_CHUNKS:
            copies[k + 1] = fire(k + 1, (k + 1) % NBUF)
        cu, cv = copies[k]
        cu.wait()
        cv.wait()
        compute(k, k % NBUF)

    pltpu.sync_copy(out_v, out_hbm.at[pl.ds(base, B_PER_W)])


@jax.jit
def _run(i3, j3, u_emb, v_emb):
    u2 = _relayout(u_emb.T)
    v2 = _relayout(v_emb.T)
    mesh = plsc.VectorSubcoreMesh(
        core_axis_name="c", subcore_axis_name="s",
        num_cores=NC, num_subcores=NS)
    f = pl.kernel(
        _sc_kernel,
        out_type=jax.ShapeDtypeStruct((BATCH,), jnp.float32),
        mesh=mesh,
        compiler_params=pltpu.CompilerParams(needs_layout_passes=False),
        scratch_types=[
            pltpu.VMEM((N_CHUNKS, CHUNK), jnp.int32),
            pltpu.VMEM((N_CHUNKS, CHUNK), jnp.int32),
            pltpu.VMEM((N_CHUNKS, CHUNK), jnp.int32),
            pltpu.VMEM((N_CHUNKS, CHUNK), jnp.int32),
            pltpu.VMEM((NBUF, CHUNK, 128), jnp.float32),
            pltpu.VMEM((NBUF, CHUNK, 128), jnp.float32),
            pltpu.VMEM((B_PER_W,), jnp.float32),
            pltpu.SemaphoreType.DMA,
        ],
    )
    return f(i3, j3, u2, v2)


def kernel(i, j, u_emb, v_emb):
    i3 = i.astype(jnp.int32).reshape(NW, N_CHUNKS, CHUNK)
    j3 = j.astype(jnp.int32).reshape(NW, N_CHUNKS, CHUNK)
    return _run(i3, j3, u_emb, v_emb)


# speedup vs baseline: 6.4769x; 1.1413x over previous
import jax
import jax.numpy as jnp
from jax import lax
from jax.experimental import pallas as pl
from jax.experimental.pallas import tpu as pltpu
from jax.experimental.pallas import tpu_sc as plsc

NC = 2
NS = 16
NW = NC * NS
LANES = 16

N_VERT = 1_000_000
BATCH = 16384
OUT_DIM = 32
PACK = 128 // OUT_DIM
L_STEP = 32768
N_STEP = -(-N_VERT // L_STEP)
N_BLK = N_STEP * (L_STEP // PACK)
B_PER_W = BATCH // NW
CHUNK = 128
N_CHUNKS = B_PER_W // CHUNK
NBUF = 2

Q = L_STEP // PACK


def _relayout_body(in_ref, out_ref):
    x = in_ref[...]
    x4 = jnp.concatenate([x[:, Q * a:Q * a + Q] for a in range(PACK)], axis=0)
    out_ref[...] = x4.T


def _relayout(tT):
    return pl.pallas_call(
        _relayout_body,
        grid=(N_STEP,),
        in_specs=[pl.BlockSpec((32, L_STEP), lambda s: (0, s))],
        out_specs=pl.BlockSpec((Q, 128), lambda s: (s, 0)),
        out_shape=jax.ShapeDtypeStruct((N_BLK, 128), jnp.float32),
    )(tT)


def _sc_kernel(i_hbm, j_hbm, u_hbm, v_hbm, out_hbm,
               raw_u, raw_v, blk_u, blk_v, u_buf, v_buf, out_v, sem):
    wid = lax.axis_index("s") * NC + lax.axis_index("c")
    base = wid * B_PER_W

    pltpu.sync_copy(i_hbm.at[wid], raw_u)
    pltpu.sync_copy(j_hbm.at[wid], raw_v)
    for k in range(N_CHUNKS):
        for t in range(CHUNK // LANES):
            s = pl.ds(t * LANES, LANES)
            ru = raw_u[k, s]
            rv = raw_v[k, s]
            blk_u[k, s] = ((ru >> 15) << 13) | (ru & (Q - 1))
            blk_v[k, s] = ((rv >> 15) << 13) | (rv & (Q - 1))

    def fire(k, buf):
        cu = pltpu.async_copy(u_hbm.at[blk_u.at[k]], u_buf.at[buf], sem)
        cv = pltpu.async_copy(v_hbm.at[blk_v.at[k]], v_buf.at[buf], sem)
        return cu, cv

    def compute(k, buf):
        def gbody(g, _):
            s = pl.ds(g * LANES, LANES)
            ru = raw_u[k, s]
            rv = raw_v[k, s]
            cu0 = ((ru >> 13) & 3) << 5
            cv0 = ((rv >> 13) & 3) << 5
            rows = g * LANES + lax.iota(jnp.int32, LANES)
            acc = jnp.zeros((LANES,), jnp.float32)
            for d in range(OUT_DIM):
                ud = plsc.load_gather(u_buf.at[buf], [rows, cu0 + d])
                vd = plsc.load_gather(v_buf.at[buf], [rows, cv0 + d])
                acc = acc + ud * vd
            out_v[pl.ds(k * CHUNK + g * LANES, LANES)] = acc
            return 0

        lax.fori_loop(0, CHUNK // LANES, gbody, 0)

    copies = [None] * N_CHUNKS
    copies[0] = fire(0, 0)
    for k in range(N_CHUNKS):
        if k + 1 < N_CHUNKS:
            copies[k + 1] = fire(k + 1, (k + 1) % NBUF)
        cu, cv = copies[k]
        cu.wait()
        cv.wait()
        compute(k, k % NBUF)

    pltpu.sync_copy(out_v, out_hbm.at[pl.ds(base, B_PER_W)])


@jax.jit
def _run(i3, j3, u_emb, v_emb):
    u2 = _relayout(u_emb.T)
    v2 = _relayout(v_emb.T)
    mesh = plsc.VectorSubcoreMesh(
        core_axis_name="c", subcore_axis_name="s",
        num_cores=NC, num_subcores=NS)
    f = pl.kernel(
        _sc_kernel,
        out_type=jax.ShapeDtypeStruct((BATCH,), jnp.float32),
        mesh=mesh,
        compiler_params=pltpu.CompilerParams(needs_layout_passes=False),
        scratch_types=[
            pltpu.VMEM((N_CHUNKS, CHUNK), jnp.int32),
            pltpu.VMEM((N_CHUNKS, CHUNK), jnp.int32),
            pltpu.VMEM((N_CHUNKS, CHUNK), jnp.int32),
            pltpu.VMEM((N_CHUNKS, CHUNK), jnp.int32),
            pltpu.VMEM((NBUF, CHUNK, 128), jnp.float32),
            pltpu.VMEM((NBUF, CHUNK, 128), jnp.float32),
            pltpu.VMEM((B_PER_W,), jnp.float32),
            pltpu.SemaphoreType.DMA,
        ],
    )
    return f(i3, j3, u2, v2)


def kernel(i, j, u_emb, v_emb):
    i3 = i.astype(jnp.int32).reshape(NW, N_CHUNKS, CHUNK)
    j3 = j.astype(jnp.int32).reshape(NW, N_CHUNKS, CHUNK)
    return _run(i3, j3, u_emb, v_emb)


# speedup vs baseline: 6.5563x; 1.0123x over previous
import jax
import jax.numpy as jnp
from jax import lax
from jax.experimental import pallas as pl
from jax.experimental.pallas import tpu as pltpu
from jax.experimental.pallas import tpu_sc as plsc

NC = 2
NS = 16
NW = NC * NS
LANES = 16

N_VERT = 1_000_000
BATCH = 16384
OUT_DIM = 32
PACK = 128 // OUT_DIM
L_STEP = 65536
N_STEP = -(-N_VERT // L_STEP)
N_BLK = N_STEP * (L_STEP // PACK)
B_PER_W = BATCH // NW
CHUNK = 128
N_CHUNKS = B_PER_W // CHUNK
NBUF = 2

Q = L_STEP // PACK


def _relayout_body(in_ref, out_ref):
    x = in_ref[...]
    x4 = jnp.concatenate([x[:, Q * a:Q * a + Q] for a in range(PACK)], axis=0)
    out_ref[...] = x4.T


def _relayout(tT):
    return pl.pallas_call(
        _relayout_body,
        grid=(N_STEP,),
        in_specs=[pl.BlockSpec((32, L_STEP), lambda s: (0, s))],
        out_specs=pl.BlockSpec((Q, 128), lambda s: (s, 0)),
        out_shape=jax.ShapeDtypeStruct((N_BLK, 128), jnp.float32),
    )(tT)


def _sc_kernel(i_hbm, j_hbm, u_hbm, v_hbm, out_hbm,
               raw_u, raw_v, blk_u, blk_v, u_buf, v_buf, out_v, sem):
    wid = lax.axis_index("s") * NC + lax.axis_index("c")
    base = wid * B_PER_W

    pltpu.sync_copy(i_hbm.at[wid], raw_u)
    pltpu.sync_copy(j_hbm.at[wid], raw_v)
    for k in range(N_CHUNKS):
        for t in range(CHUNK // LANES):
            s = pl.ds(t * LANES, LANES)
            ru = raw_u[k, s]
            rv = raw_v[k, s]
            blk_u[k, s] = ((ru >> 16) << 14) | (ru & (Q - 1))
            blk_v[k, s] = ((rv >> 16) << 14) | (rv & (Q - 1))

    def fire(k, buf):
        cu = pltpu.async_copy(u_hbm.at[blk_u.at[k]], u_buf.at[buf], sem)
        cv = pltpu.async_copy(v_hbm.at[blk_v.at[k]], v_buf.at[buf], sem)
        return cu, cv

    def compute(k, buf):
        def gbody(g, _):
            s = pl.ds(g * LANES, LANES)
            ru = raw_u[k, s]
            rv = raw_v[k, s]
            cu0 = ((ru >> 14) & 3) << 5
            cv0 = ((rv >> 14) & 3) << 5
            rows = g * LANES + lax.iota(jnp.int32, LANES)
            acc = jnp.zeros((LANES,), jnp.float32)
            for d in range(OUT_DIM):
                ud = plsc.load_gather(u_buf.at[buf], [rows, cu0 + d])
                vd = plsc.load_gather(v_buf.at[buf], [rows, cv0 + d])
                acc = acc + ud * vd
            out_v[pl.ds(k * CHUNK + g * LANES, LANES)] = acc
            return 0

        lax.fori_loop(0, CHUNK // LANES, gbody, 0)

    copies = [None] * N_CHUNKS
    copies[0] = fire(0, 0)
    for k in range(N_CHUNKS):
        if k + 1 < N_CHUNKS:
            copies[k + 1] = fire(k + 1, (k + 1) % NBUF)
        cu, cv = copies[k]
        cu.wait()
        cv.wait()
        compute(k, k % NBUF)

    pltpu.sync_copy(out_v, out_hbm.at[pl.ds(base, B_PER_W)])


@jax.jit
def _run(i3, j3, u_emb, v_emb):
    u2 = _relayout(u_emb.T)
    v2 = _relayout(v_emb.T)
    mesh = plsc.VectorSubcoreMesh(
        core_axis_name="c", subcore_axis_name="s",
        num_cores=NC, num_subcores=NS)
    f = pl.kernel(
        _sc_kernel,
        out_type=jax.ShapeDtypeStruct((BATCH,), jnp.float32),
        mesh=mesh,
        compiler_params=pltpu.CompilerParams(needs_layout_passes=False),
        scratch_types=[
            pltpu.VMEM((N_CHUNKS, CHUNK), jnp.int32),
            pltpu.VMEM((N_CHUNKS, CHUNK), jnp.int32),
            pltpu.VMEM((N_CHUNKS, CHUNK), jnp.int32),
            pltpu.VMEM((N_CHUNKS, CHUNK), jnp.int32),
            pltpu.VMEM((NBUF, CHUNK, 128), jnp.float32),
            pltpu.VMEM((NBUF, CHUNK, 128), jnp.float32),
            pltpu.VMEM((B_PER_W,), jnp.float32),
            pltpu.SemaphoreType.DMA,
        ],
    )
    return f(i3, j3, u2, v2)


def kernel(i, j, u_emb, v_emb):
    i3 = i.astype(jnp.int32).reshape(NW, N_CHUNKS, CHUNK)
    j3 = j.astype(jnp.int32).reshape(NW, N_CHUNKS, CHUNK)
    return _run(i3, j3, u_emb, v_emb)


# speedup vs baseline: 6.5720x; 1.0024x over previous
import jax
import jax.numpy as jnp
from jax import lax
from jax.experimental import pallas as pl
from jax.experimental.pallas import tpu as pltpu
from jax.experimental.pallas import tpu_sc as plsc

NC = 2
NS = 16
NW = NC * NS
LANES = 16

N_VERT = 1_000_000
BATCH = 16384
OUT_DIM = 32
PACK = 128 // OUT_DIM
L_STEP = 65536
N_STEP = -(-N_VERT // L_STEP)
N_BLK = N_STEP * (L_STEP // PACK)
B_PER_W = BATCH // NW
CHUNK = 128
N_CHUNKS = B_PER_W // CHUNK
NBUF = 2

Q = L_STEP // PACK
LOG_L = L_STEP.bit_length() - 1
LOG_Q = Q.bit_length() - 1


def _relayout_body(in_ref, out_ref):
    x = in_ref[...]
    x4 = jnp.concatenate([x[:, Q * a:Q * a + Q] for a in range(PACK)], axis=0)
    out_ref[...] = x4.T


def _relayout(tT):
    return pl.pallas_call(
        _relayout_body,
        grid=(N_STEP,),
        in_specs=[pl.BlockSpec((32, L_STEP), lambda s: (0, s))],
        out_specs=pl.BlockSpec((Q, 128), lambda s: (s, 0)),
        out_shape=jax.ShapeDtypeStruct((N_BLK, 128), jnp.float32),
    )(tT)


def _sc_kernel(i_hbm, j_hbm, u_hbm, v_hbm, out_hbm,
               raw_u, raw_v, blk_u, blk_v, u_buf, v_buf, out_v, sem):
    wid = lax.axis_index("s") * NC + lax.axis_index("c")
    base = wid * B_PER_W

    pltpu.sync_copy(i_hbm.at[wid], raw_u)
    pltpu.sync_copy(j_hbm.at[wid], raw_v)
    for k in range(N_CHUNKS):
        for t in range(CHUNK // LANES):
            s = pl.ds(t * LANES, LANES)
            ru = raw_u[k, s]
            rv = raw_v[k, s]
            blk_u[k, s] = ((ru >> LOG_L) << LOG_Q) | (ru & (Q - 1))
            blk_v[k, s] = ((rv >> LOG_L) << LOG_Q) | (rv & (Q - 1))

    def fire(k, buf):
        cu = pltpu.async_copy(u_hbm.at[blk_u.at[k]], u_buf.at[buf], sem)
        cv = pltpu.async_copy(v_hbm.at[blk_v.at[k]], v_buf.at[buf], sem)
        return cu, cv

    def compute(k, buf):
        def gbody(g, _):
            s = pl.ds(g * LANES, LANES)
            ru = raw_u[k, s]
            rv = raw_v[k, s]
            cu0 = ((ru >> LOG_Q) & (PACK - 1)) << 5
            cv0 = ((rv >> LOG_Q) & (PACK - 1)) << 5
            rows = g * LANES + lax.iota(jnp.int32, LANES)
            acc = jnp.zeros((LANES,), jnp.float32)
            for d in range(OUT_DIM):
                ud = plsc.load_gather(u_buf.at[buf], [rows, cu0 + d])
                vd = plsc.load_gather(v_buf.at[buf], [rows, cv0 + d])
                acc = acc + ud * vd
            out_v[pl.ds(k * CHUNK + g * LANES, LANES)] = acc
            return 0

        lax.fori_loop(0, CHUNK // LANES, gbody, 0)

    copies = [None] * N_CHUNKS
    copies[0] = fire(0, 0)
    for k in range(N_CHUNKS):
        if k + 1 < N_CHUNKS:
            copies[k + 1] = fire(k + 1, (k + 1) % NBUF)
        cu, cv = copies[k]
        cu.wait()
        cv.wait()
        compute(k, k % NBUF)

    pltpu.sync_copy(out_v, out_hbm.at[pl.ds(base, B_PER_W)])


@jax.jit
def _run(i3, j3, u_emb, v_emb):
    u2 = _relayout(u_emb.T)
    v2 = _relayout(v_emb.T)
    mesh = plsc.VectorSubcoreMesh(
        core_axis_name="c", subcore_axis_name="s",
        num_cores=NC, num_subcores=NS)
    f = pl.kernel(
        _sc_kernel,
        out_type=jax.ShapeDtypeStruct((BATCH,), jnp.float32),
        mesh=mesh,
        compiler_params=pltpu.CompilerParams(needs_layout_passes=False),
        scratch_types=[
            pltpu.VMEM((N_CHUNKS, CHUNK), jnp.int32),
            pltpu.VMEM((N_CHUNKS, CHUNK), jnp.int32),
            pltpu.VMEM((N_CHUNKS, CHUNK), jnp.int32),
            pltpu.VMEM((N_CHUNKS, CHUNK), jnp.int32),
            pltpu.VMEM((NBUF, CHUNK, 128), jnp.float32),
            pltpu.VMEM((NBUF, CHUNK, 128), jnp.float32),
            pltpu.VMEM((B_PER_W,), jnp.float32),
            pltpu.SemaphoreType.DMA,
        ],
    )
    return f(i3, j3, u2, v2)


def kernel(i, j, u_emb, v_emb):
    i3 = i.astype(jnp.int32).reshape(NW, N_CHUNKS, CHUNK)
    j3 = j.astype(jnp.int32).reshape(NW, N_CHUNKS, CHUNK)
    return _run(i3, j3, u_emb, v_emb)
